# baseline (speedup 1.0000x reference)
import functools

import jax
import jax.numpy as jnp
from jax import lax
from jax.experimental import pallas as pl
from jax.experimental.pallas import tpu as pltpu
from jax.experimental.pallas import tpu_sc as plsc

N = 10000
E = 320000
D = 128
L = 4

NC = 2
NS = 16
NW = NC * NS

IDXW = 128
ROWS_PT = 80
EPT = IDXW * ROWS_PT
E_PAD = NW * EPT

CH = 64
NCH = EPT // CH
NSEC = 4
SEC = NCH // NSEC

N_PAD = 10112
RPT = N_PAD // NS
DEGW = 128

BLK = 1000

_mesh = plsc.VectorSubcoreMesh(core_axis_name="c", subcore_axis_name="s")


@functools.partial(
    pl.kernel,
    out_type=jax.ShapeDtypeStruct((NC, N_PAD, D), jnp.float32),
    mesh=_mesh,
    scratch_types=[
        pltpu.VMEM((SEC, CH), jnp.int32),
        pltpu.VMEM((SEC, CH), jnp.int32),
        pltpu.VMEM((CH, D), jnp.float32),
        pltpu.VMEM((CH, D), jnp.float32),
        pltpu.VMEM((CH, D), jnp.float32),
        pltpu.VMEM((CH, D), jnp.float32),
        pltpu.VMEM_SHARED((N_PAD, D), jnp.float32),
        pltpu.SemaphoreType.DMA,
        pltpu.SemaphoreType.DMA,
        pltpu.SemaphoreType.DMA,
        pltpu.SemaphoreType.DMA,
        pltpu.SemaphoreType.DMA,
        pltpu.SemaphoreType.DMA,
        pltpu.SemaphoreType.DMA,
        pltpu.SemaphoreType.DMA,
    ],
)
def _sc_segsum(h_hbm, src_hbm, dst_hbm, z_hbm, out_hbm,
               sidx, didx, b0, b1, b2, b3, acc,
               g0, g1, g2, g3, s0, s1, s2, s3):
    c = lax.axis_index("c")
    s = lax.axis_index("s")
    base = (c * NS + s) * NCH
    pltpu.sync_copy(z_hbm, acc.at[pl.ds(s * RPT, RPT)])
    plsc.subcore_barrier()

    bufs = (b0, b1, b2, b3)
    gsem = (g0, g1, g2, g3)
    ssem = (s0, s1, s2, s3)

    def gather(t, b):
        pltpu.async_copy(h_hbm.at[sidx.at[t]], bufs[b], gsem[b])

    def gw(t, b):
        pltpu.make_async_copy(h_hbm.at[sidx.at[t]], bufs[b], gsem[b]).wait()

    def scat(t, b):
        pltpu.async_copy(bufs[b], acc.at[didx.at[t]], ssem[b], add=True)

    def scw(t, b):
        pltpu.make_async_copy(bufs[b], acc.at[didx.at[t]], ssem[b]).wait()

    for hf in range(NSEC):
        pltpu.sync_copy(src_hbm.at[pl.ds(base + hf * SEC, SEC)], sidx)
        pltpu.sync_copy(dst_hbm.at[pl.ds(base + hf * SEC, SEC)], didx)

        gather(0, 0)
        gather(1, 1)
        gw(0, 0)
        scat(0, 0)
        gw(1, 1)
        scat(1, 1)
        gather(2, 2)
        gather(3, 3)

        @pl.loop(1, SEC // 2 - 1, step=2)
        def _(kk):
            t = 2 * kk
            gw(t, 2)
            scat(t, 2)
            gw(t + 1, 3)
            scat(t + 1, 3)
            scw(t - 2, 0)
            gather(t + 2, 0)
            scw(t - 1, 1)
            gather(t + 3, 1)
            gw(t + 2, 0)
            scat(t + 2, 0)
            gw(t + 3, 1)
            scat(t + 3, 1)
            scw(t, 2)
            gather(t + 4, 2)
            scw(t + 1, 3)
            gather(t + 5, 3)

        gw(SEC - 2, 2)
        scat(SEC - 2, 2)
        gw(SEC - 1, 3)
        scat(SEC - 1, 3)
        scw(SEC - 4, 0)
        scw(SEC - 3, 1)
        scw(SEC - 2, 2)
        scw(SEC - 1, 3)

    plsc.subcore_barrier()
    pltpu.sync_copy(acc.at[pl.ds(s * RPT, RPT)],
                    out_hbm.at[c, pl.ds(s * RPT, RPT)])


@functools.partial(
    pl.kernel,
    out_type=jax.ShapeDtypeStruct((NC, N_PAD, DEGW), jnp.float32),
    mesh=_mesh,
    scratch_types=[
        pltpu.VMEM((ROWS_PT // 2, IDXW), jnp.int32),
        pltpu.VMEM((IDXW, DEGW), jnp.float32),
        pltpu.VMEM_SHARED((N_PAD, DEGW), jnp.float32),
        pltpu.SemaphoreType.DMA,
        pltpu.SemaphoreType.DMA,
    ],
)
def _sc_degree(dst_hbm, ones_hbm, z_hbm, out_hbm, didx, ones_v, acc,
               sg0, sg1):
    c = lax.axis_index("c")
    s = lax.axis_index("s")
    pltpu.sync_copy(z_hbm, acc.at[pl.ds(s * RPT, RPT)])
    pltpu.sync_copy(ones_hbm, ones_v)
    plsc.subcore_barrier()

    base = (c * NS + s) * ROWS_PT
    HALF = ROWS_PT // 2

    def scat(t, sem):
        pltpu.async_copy(ones_v, acc.at[didx.at[t]], sem, add=True)

    def scw(t, sem):
        pltpu.make_async_copy(ones_v, acc.at[didx.at[t]], sem).wait()

    for hf in range(2):
        pltpu.sync_copy(dst_hbm.at[pl.ds(base + hf * HALF, HALF)], didx)
        scat(0, sg0)
        scat(1, sg1)

        @pl.loop(0, HALF - 2, step=2)
        def _(t):
            scw(t, sg0)
            scat(t + 2, sg0)
            scw(t + 1, sg1)
            scat(t + 3, sg1)

        scw(HALF - 2, sg0)
        scw(HALF - 1, sg1)

    plsc.subcore_barrier()
    pltpu.sync_copy(acc.at[pl.ds(s * RPT, RPT)],
                    out_hbm.at[c, pl.ds(s * RPT, RPT)])


def _tc_proj(x, W, b):
    def body(x_ref, w_ref, b_ref, o_ref):
        o_ref[...] = (
            jnp.dot(x_ref[...], w_ref[...], preferred_element_type=jnp.float32)
            + b_ref[...]
        )

    return pl.pallas_call(
        body,
        grid=(N // BLK,),
        in_specs=[
            pl.BlockSpec((BLK, D), lambda i: (i, 0)),
            pl.BlockSpec((D, D), lambda i: (0, 0)),
            pl.BlockSpec((1, D), lambda i: (0, 0)),
        ],
        out_specs=pl.BlockSpec((BLK, D), lambda i: (i, 0)),
        out_shape=jax.ShapeDtypeStruct((N, D), jnp.float32),
    )(x, W, b)


def _tc_layer(parts, deg_parts, h, Wl_i, Wr_i, bl_i, g_i, beta_i):
    def body(p_ref, dp_ref, h_ref, wl_ref, wr_ref, bl_ref, g_ref, be_ref,
             o_ref):
        deg = jnp.maximum(dp_ref[0, :, 0:1] + dp_ref[1, :, 0:1], 1.0)
        msg = (p_ref[0] + p_ref[1]) / deg
        out = (
            jnp.dot(msg, wl_ref[...], preferred_element_type=jnp.float32)
            + bl_ref[...]
            + jnp.dot(h_ref[...], wr_ref[...],
                      preferred_element_type=jnp.float32)
        )
        out = out * jax.nn.sigmoid(out)
        mu = jnp.mean(out, axis=1, keepdims=True)
        var = jnp.mean((out - mu) ** 2, axis=1, keepdims=True)
        o_ref[...] = (out - mu) * lax.rsqrt(var + 1e-5) * g_ref[...] \
            + be_ref[...]

    return pl.pallas_call(
        body,
        grid=(N // BLK,),
        in_specs=[
            pl.BlockSpec((NC, BLK, D), lambda i: (0, i, 0)),
            pl.BlockSpec((NC, BLK, DEGW), lambda i: (0, i, 0)),
            pl.BlockSpec((BLK, D), lambda i: (i, 0)),
            pl.BlockSpec((D, D), lambda i: (0, 0)),
            pl.BlockSpec((D, D), lambda i: (0, 0)),
            pl.BlockSpec((1, D), lambda i: (0, 0)),
            pl.BlockSpec((1, D), lambda i: (0, 0)),
            pl.BlockSpec((1, D), lambda i: (0, 0)),
        ],
        out_specs=pl.BlockSpec((BLK, D), lambda i: (i, 0)),
        out_shape=jax.ShapeDtypeStruct((N, D), jnp.float32),
    )(parts, deg_parts, h, Wl_i, Wr_i, bl_i, g_i, beta_i)


def kernel(x, edge_index, W_in, b_in, Wl, bl, Wr, g, beta):
    src = edge_index[0]
    dst = edge_index[1]
    npad = E_PAD - E
    pad_src = jnp.arange(npad, dtype=jnp.int32) % N
    pad_dst = N + jnp.arange(npad, dtype=jnp.int32) % (N_PAD - N)
    src_full = jnp.concatenate([src, pad_src])
    dst_full = jnp.concatenate([dst, pad_dst])
    src2 = src_full.reshape(E_PAD // CH, CH)
    dst2 = dst_full.reshape(E_PAD // CH, CH)
    dst2_deg = dst_full.reshape(E_PAD // IDXW, IDXW)

    zeros_msg = jnp.zeros((RPT, D), jnp.float32)
    zeros_deg = jnp.zeros((RPT, DEGW), jnp.float32)
    ones_deg = jnp.ones((IDXW, DEGW), jnp.float32)

    deg_parts = _sc_degree(dst2_deg, ones_deg, zeros_deg)
    h = _tc_proj(x, W_in, b_in.reshape(1, D))
    for i in range(L):
        parts = _sc_segsum(h, src2, dst2, zeros_msg)
        h = _tc_layer(parts, deg_parts, h, Wl[i], Wr[i],
                      bl[i].reshape(1, D), g[i].reshape(1, D),
                      beta[i].reshape(1, D))
    return h
